# baseline (speedup 1.0000x reference)
import functools

import jax
import jax.numpy as jnp
from jax.experimental import pallas as pl
from jax.experimental.pallas import tpu as pltpu


def _project_body(table_ref, w_ref, b_ref, out_ref):
    out_ref[...] = (
        jnp.dot(table_ref[...], w_ref[...], preferred_element_type=jnp.float32)
        + b_ref[...]
    )


def _project(table, W, b):
    V, _ = table.shape
    MD = W.shape[1]
    return pl.pallas_call(
        _project_body,
        out_shape=jax.ShapeDtypeStruct((V, MD), jnp.float32),
    )(table, W, b.reshape(1, MD))


def _make_row_gather(V, MD, B, L, CB):
    n_steps = B // CB

    def body(idx_ref, p_hbm, out_hbm, p_ref, buf0, buf1, sem_p, s0, s1):
        step = pl.program_id(0)

        @pl.when(step == 0)
        def _():
            cp = pltpu.make_async_copy(p_hbm, p_ref, sem_p)
            cp.start()
            cp.wait()

        bufs = (buf0, buf1)
        sems = (s0, s1)

        def run(par):
            buf = bufs[par]
            sem = sems[par]

            @pl.when(step >= 2)
            def _():
                pltpu.make_async_copy(
                    buf, out_hbm.at[pl.ds(0, CB)], sem
                ).wait()

            for bb in range(CB):
                for l in range(L):
                    i = idx_ref[0, bb, l]
                    buf[bb, l, :] = p_ref[i, :]

            pltpu.async_copy(buf, out_hbm.at[pl.ds(step * CB, CB)], sem)

            @pl.when(step == n_steps - 1)
            def _():
                pltpu.make_async_copy(
                    bufs[1 - par], out_hbm.at[pl.ds(0, CB)], sems[1 - par]
                ).wait()
                pltpu.make_async_copy(
                    buf, out_hbm.at[pl.ds(0, CB)], sem
                ).wait()

        @pl.when(step % 2 == 0)
        def _():
            run(0)

        @pl.when(step % 2 == 1)
        def _():
            run(1)

    return pl.pallas_call(
        body,
        grid=(n_steps,),
        in_specs=[
            pl.BlockSpec((1, CB, L), lambda b: (b, 0, 0),
                         memory_space=pltpu.SMEM),
            pl.BlockSpec(memory_space=pl.ANY),
        ],
        out_specs=pl.BlockSpec(memory_space=pl.ANY),
        out_shape=jax.ShapeDtypeStruct((B, L, MD), jnp.float32),
        scratch_shapes=[
            pltpu.VMEM((V, MD), jnp.float32),
            pltpu.VMEM((CB, L, MD), jnp.float32),
            pltpu.VMEM((CB, L, MD), jnp.float32),
            pltpu.SemaphoreType.DMA,
            pltpu.SemaphoreType.DMA,
            pltpu.SemaphoreType.DMA,
        ],
        compiler_params=pltpu.CompilerParams(
            dimension_semantics=("arbitrary",)
        ),
    )


def kernel(x, table, W, b):
    B, L = x.shape
    V, D = table.shape
    MD = W.shape[1]
    P = _project(table, W, b)
    CB = 16
    x3 = x.astype(jnp.int32).reshape(B // CB, CB, L)
    return _make_row_gather(V, MD, B, L, CB)(x3, P)


# speedup vs baseline: 1.1700x; 1.1700x over previous
import functools

import jax
import jax.numpy as jnp
from jax.experimental import pallas as pl
from jax.experimental.pallas import tpu as pltpu


def _project_body(table_ref, w_ref, b_ref, out_ref):
    p = (
        jnp.dot(table_ref[...], w_ref[...], preferred_element_type=jnp.float32)
        + b_ref[...]
    )
    out_ref[...] = p.reshape(out_ref.shape)


def _project(table, W, b):
    V, _ = table.shape
    MD = W.shape[1]
    return pl.pallas_call(
        _project_body,
        out_shape=jax.ShapeDtypeStruct((V, 8, MD // 8), jnp.float32),
    )(table, W, b.reshape(1, MD))


def _make_row_gather(V, MD, B, L, TB):
    def body(idx_ref, p_hbm, out_ref, p_ref, sem):
        @pl.when(pl.program_id(0) == 0)
        def _():
            cp = pltpu.make_async_copy(p_hbm, p_ref, sem)
            cp.start()
            cp.wait()

        for bb in range(TB):
            for tr in range(L // 8):
                rows8 = jnp.stack(
                    [p_ref[idx_ref[0, bb, 8 * tr + s]] for s in range(8)],
                    axis=0,
                )
                out_ref[bb, 8 * tr:8 * tr + 8, :] = rows8.reshape(8, MD)
            rem = L % 8
            if rem:
                tr = L // 8
                rows_r = jnp.stack(
                    [p_ref[idx_ref[0, bb, 8 * tr + s]] for s in range(rem)],
                    axis=0,
                )
                out_ref[bb, 8 * tr:L, :] = rows_r.reshape(rem, MD)

    return pl.pallas_call(
        body,
        grid=(B // TB,),
        in_specs=[
            pl.BlockSpec((1, TB, L), lambda b: (b, 0, 0),
                         memory_space=pltpu.SMEM),
            pl.BlockSpec(memory_space=pl.ANY),
        ],
        out_specs=pl.BlockSpec((TB, L, MD), lambda b: (b, 0, 0)),
        out_shape=jax.ShapeDtypeStruct((B, L, MD), jnp.float32),
        scratch_shapes=[
            pltpu.VMEM((V, 8, MD // 8), jnp.float32),
            pltpu.SemaphoreType.DMA,
        ],
        compiler_params=pltpu.CompilerParams(
            dimension_semantics=("parallel",)
        ),
    )


def kernel(x, table, W, b):
    B, L = x.shape
    V, D = table.shape
    MD = W.shape[1]
    P8 = _project(table, W, b)
    TB = 32
    x3 = x.astype(jnp.int32).reshape(B // TB, TB, L)
    return _make_row_gather(V, MD, B, L, TB)(x3, P8)


# speedup vs baseline: 1.1897x; 1.0168x over previous
import functools

import jax
import jax.numpy as jnp
from jax.experimental import pallas as pl
from jax.experimental.pallas import tpu as pltpu


def _project_body(table_ref, w_ref, b_ref, out_ref):
    p = (
        jnp.dot(table_ref[...], w_ref[...], preferred_element_type=jnp.float32)
        + b_ref[...]
    )
    out_ref[...] = p.reshape(out_ref.shape)


def _project(table, W, b):
    V, _ = table.shape
    MD = W.shape[1]
    return pl.pallas_call(
        _project_body,
        out_shape=jax.ShapeDtypeStruct((V, 8, MD // 8), jnp.float32),
    )(table, W, b.reshape(1, MD))


def _make_row_gather(V, MD, B, L, TB):
    def body(idx_ref, p_hbm, out_ref, p_ref, sem):
        @pl.when(pl.program_id(0) == 0)
        def _():
            cp = pltpu.make_async_copy(p_hbm, p_ref, sem)
            cp.start()
            cp.wait()

        for bb in range(TB):
            for tr in range(L // 8):
                rows8 = jnp.stack(
                    [p_ref[idx_ref[0, bb, 8 * tr + s]] for s in range(8)],
                    axis=0,
                )
                out_ref[bb, 8 * tr:8 * tr + 8, :] = rows8.reshape(8, MD)
            rem = L % 8
            if rem:
                tr = L // 8
                rows_r = jnp.stack(
                    [p_ref[idx_ref[0, bb, 8 * tr + s]] for s in range(rem)],
                    axis=0,
                )
                out_ref[bb, 8 * tr:L, :] = rows_r.reshape(rem, MD)

    return pl.pallas_call(
        body,
        grid=(B // TB,),
        in_specs=[
            pl.BlockSpec((1, TB, L), lambda b: (b, 0, 0),
                         memory_space=pltpu.SMEM),
            pl.BlockSpec(memory_space=pl.ANY),
        ],
        out_specs=pl.BlockSpec((TB, L, MD), lambda b: (b, 0, 0)),
        out_shape=jax.ShapeDtypeStruct((B, L, MD), jnp.float32),
        scratch_shapes=[
            pltpu.VMEM((V, 8, MD // 8), jnp.float32),
            pltpu.SemaphoreType.DMA,
        ],
        compiler_params=pltpu.CompilerParams(
            dimension_semantics=("parallel",)
        ),
    )


def kernel(x, table, W, b):
    B, L = x.shape
    V, D = table.shape
    MD = W.shape[1]
    P8 = _project(table, W, b)
    TB = 64
    x3 = x.astype(jnp.int32).reshape(B // TB, TB, L)
    return _make_row_gather(V, MD, B, L, TB)(x3, P8)
